# baseline (speedup 1.0000x reference)
import functools

import jax
import jax.numpy as jnp
from jax import lax
from jax.experimental import pallas as pl
from jax.experimental.pallas import tpu as pltpu
from jax.experimental.pallas import tpu_sc as plsc

B = 16384
D = 64
WIDE = 2 * D
NROW_W = 500000
NC = 2
NS = 16
NW = NC * NS
B_PER_W = B // NW
ICH = 128
N_ICH = B_PER_W // ICH


def _sc_gather_body(wide_hbm, idx_hbm, out_hbm, idx_v, tid_v, rows_v, sem):
    wid = lax.axis_index("s") * NC + lax.axis_index("c")
    base = wid * B_PER_W
    pltpu.sync_copy(idx_hbm.at[pl.ds(base, B_PER_W)], idx_v)
    for g in range(B_PER_W // 16):
        v = idx_v[pl.ds(16 * g, 16)]
        tid_v[pl.ds(16 * g, 16)] = lax.shift_right_logical(v, 1)
    copies = []
    for j in range(N_ICH):
        copies.append(
            pltpu.async_copy(
                wide_hbm.at[tid_v.at[pl.ds(ICH * j, ICH)]],
                rows_v.at[pl.ds(ICH * j, ICH)],
                sem,
            )
        )
    for c in copies:
        c.wait()
    pltpu.sync_copy(rows_v, out_hbm.at[pl.ds(base, B_PER_W)])


@functools.cache
def _sc_gather():
    return pl.kernel(
        _sc_gather_body,
        out_type=jax.ShapeDtypeStruct((B, WIDE), jnp.float32),
        mesh=plsc.VectorSubcoreMesh(core_axis_name="c", subcore_axis_name="s"),
        scratch_types=[
            pltpu.VMEM((B_PER_W,), jnp.int32),
            pltpu.VMEM((B_PER_W,), jnp.int32),
            pltpu.VMEM((B_PER_W, WIDE), jnp.float32),
            pltpu.SemaphoreType.DMA,
        ],
        compiler_params=pltpu.CompilerParams(needs_layout_passes=False),
    )


def _tc_loss_body(x_ref, w_ref, p_ref, out_ref):
    i = pl.program_id(0)
    x = x_ref[...]
    par = p_ref[...]
    g = w_ref[:, :D] * (1.0 - par) + w_ref[:, D:] * par
    m = jnp.max(x, axis=1, keepdims=True)
    e = jnp.exp(x - m)
    z = jnp.sum(e, axis=1, keepdims=True)
    logsm = x - m - jnp.log(z)
    part = -jnp.sum(logsm * g, keepdims=True) * (1.0 / B)

    @pl.when(i == 0)
    def _init():
        out_ref[...] = part

    @pl.when(i != 0)
    def _acc():
        out_ref[...] += part


_N_BLK = 8
_BLK = B // _N_BLK

_tc_loss = pl.pallas_call(
    _tc_loss_body,
    grid=(_N_BLK,),
    in_specs=[
        pl.BlockSpec((_BLK, D), lambda i: (i, 0)),
        pl.BlockSpec((_BLK, WIDE), lambda i: (i, 0)),
        pl.BlockSpec((_BLK, 1), lambda i: (i, 0)),
    ],
    out_specs=pl.BlockSpec((1, 1), lambda i: (0, 0)),
    out_shape=jax.ShapeDtypeStruct((1, 1), jnp.float32),
)


def kernel(outputs, index, confidence):
    idx = index.astype(jnp.int32)
    wide = confidence.reshape(NROW_W, WIDE)
    rows = _sc_gather()(wide, idx)
    par = (idx & 1).astype(jnp.float32).reshape(B, 1)
    loss = _tc_loss(outputs, rows, par)
    return loss[0, 0]
